# baseline (speedup 1.0000x reference)
import jax
import jax.numpy as jnp
from jax.experimental import pallas as pl
from jax.experimental.pallas import tpu as pltpu

_TILE = 1024
_K = 1024
_D = 32
_CW = 0.25


def _vq_kernel(x_ref, e_ref, q_ref, idx_ref, acc_ref):
    x = x_ref[...]
    e = e_ref[...]
    x2 = jnp.sum(x * x, axis=1, keepdims=True)
    e2 = jnp.sum(e * e, axis=1, keepdims=True).T
    xe = jax.lax.dot_general(x, e, (((1,), (1,)), ((), ())),
                             preferred_element_type=jnp.float32)
    dist = (x2 - 2.0 * xe) + e2
    min_d = jnp.min(dist, axis=1, keepdims=True)
    lane = jax.lax.broadcasted_iota(jnp.int32, dist.shape, 1)
    idx = jnp.min(jnp.where(dist == min_d, lane, _K), axis=1)
    onehot = (lane == idx[:, None]).astype(jnp.float32)
    q = jax.lax.dot_general(onehot, e, (((1,), (0,)), ((), ())),
                            precision=jax.lax.Precision.HIGHEST,
                            preferred_element_type=jnp.float32)
    q_ref[...] = x + (q - x)
    idx_ref[0, 0, :] = idx
    diff = x - q
    part = jnp.sum(diff * diff)

    @pl.when(pl.program_id(0) == 0)
    def _init():
        acc_ref[0, 0] = 0.0

    acc_ref[0, 0] += part


def kernel(x, embed):
    B, T, D = x.shape
    xf = x.reshape(-1, D)
    n = xf.shape[0]
    g = n // _TILE
    q, idx3, acc = pl.pallas_call(
        _vq_kernel,
        grid=(g,),
        in_specs=[pl.BlockSpec((_TILE, D), lambda i: (i, 0)),
                  pl.BlockSpec((_K, D), lambda i: (0, 0))],
        out_specs=[pl.BlockSpec((_TILE, D), lambda i: (i, 0)),
                   pl.BlockSpec((1, 1, _TILE), lambda i: (i, 0, 0)),
                   pl.BlockSpec((1, 1), lambda i: (0, 0),
                                memory_space=pltpu.SMEM)],
        out_shape=[jax.ShapeDtypeStruct((n, D), jnp.float32),
                   jax.ShapeDtypeStruct((g, 1, _TILE), jnp.int32),
                   jax.ShapeDtypeStruct((1, 1), jnp.float32)],
    )(xf, embed)
    loss = (1.0 + _CW) * acc[0, 0] / (n * D)
    return q.reshape(B, T, D), idx3.reshape(B, T), loss
